# baseline (speedup 1.0000x reference)
import functools

import jax
import jax.numpy as jnp
from jax import lax
from jax.experimental import pallas as pl
from jax.experimental.pallas import tpu as pltpu
from jax.experimental.pallas import tpu_sc as plsc

B = 1024
IN_DIM = 4096
OUT_DIM = 4096
FAN_IN = 8
L = 16
NC, NS = 2, 16
NW = NC * NS
RPW = B // NW
G = OUT_DIM // L


def _tec_body(x_hbm, idxT_hbm, wT_hbm, s_hbm, b_hbm, out_hbm,
              xrow, yrow, idxv, wv, sv, bv):
    wid = lax.axis_index("s") * NC + lax.axis_index("c")
    base = wid * RPW
    pltpu.sync_copy(idxT_hbm, idxv)
    pltpu.sync_copy(wT_hbm, wv)
    pltpu.sync_copy(s_hbm, sv)
    pltpu.sync_copy(b_hbm, bv)

    def row_body(r, carry):
        row = base + r
        pltpu.sync_copy(x_hbm.at[row], xrow)

        def grp_body(g, c2):
            o = g * L
            acc = plsc.load_gather(xrow, [idxv[0, pl.ds(o, L)]]) \
                * wv[0, pl.ds(o, L)]
            for k in range(1, FAN_IN):
                acc = acc + plsc.load_gather(xrow, [idxv[k, pl.ds(o, L)]]) \
                    * wv[k, pl.ds(o, L)]
            z = acc * sv[pl.ds(o, L)] - bv[pl.ds(o, L)]
            yrow[pl.ds(o, L)] = 1.0 / (1.0 + jnp.exp(-z))
            return c2

        lax.fori_loop(0, G, grp_body, 0)
        pltpu.sync_copy(yrow, out_hbm.at[row])
        return carry

    lax.fori_loop(0, RPW, row_body, 0)


def kernel(x, idx, w, theta, s_raw):
    idxT = jnp.asarray(idx, jnp.int32).T
    wT = w.T
    s = jax.nn.softplus(s_raw) + 1e-6
    bterm = s * theta

    mesh = plsc.VectorSubcoreMesh(core_axis_name="c", subcore_axis_name="s")
    run = functools.partial(
        pl.kernel,
        mesh=mesh,
        compiler_params=pltpu.CompilerParams(needs_layout_passes=False),
        out_type=jax.ShapeDtypeStruct((B, OUT_DIM), jnp.float32),
        scratch_types=[
            pltpu.VMEM((IN_DIM,), jnp.float32),
            pltpu.VMEM((OUT_DIM,), jnp.float32),
            pltpu.VMEM((FAN_IN, OUT_DIM), jnp.int32),
            pltpu.VMEM((FAN_IN, OUT_DIM), jnp.float32),
            pltpu.VMEM((OUT_DIM,), jnp.float32),
            pltpu.VMEM((OUT_DIM,), jnp.float32),
        ],
    )(_tec_body)
    return run(x, idxT, wT, s, bterm)


# speedup vs baseline: 2.2133x; 2.2133x over previous
import functools

import jax
import jax.numpy as jnp
from jax import lax
from jax.experimental import pallas as pl
from jax.experimental.pallas import tpu as pltpu
from jax.experimental.pallas import tpu_sc as plsc

B = 1024
IN_DIM = 4096
OUT_DIM = 4096
FAN_IN = 8
L = 16
NC, NS = 2, 16
NW = NC * NS
RPW = B // NW
G = OUT_DIM // L
RB = 4
NPASS = RPW // RB


def _tec_body(x_hbm, idxT_hbm, wT_hbm, s_hbm, b_hbm, out_hbm,
              x0, x1, x2, x3, y0, y1, y2, y3, idxv, wv, sv, bv, sem):
    xr = (x0, x1, x2, x3)
    yr = (y0, y1, y2, y3)
    wid = lax.axis_index("s") * NC + lax.axis_index("c")
    base = wid * RPW
    pltpu.sync_copy(idxT_hbm, idxv)
    pltpu.sync_copy(wT_hbm, wv)
    pltpu.sync_copy(s_hbm, sv)
    pltpu.sync_copy(b_hbm, bv)

    def pass_body(p, carry):
        row = base + p * RB
        cps = [pltpu.async_copy(x_hbm.at[row + r], xr[r], sem)
               for r in range(RB)]
        for c in cps:
            c.wait()

        def grp_body(g, c2):
            o = g * L
            acc = [None] * RB
            for k in range(FAN_IN):
                ivec = idxv[k, pl.ds(o, L)]
                wvec = wv[k, pl.ds(o, L)]
                for r in range(RB):
                    t = plsc.load_gather(xr[r], [ivec]) * wvec
                    acc[r] = t if k == 0 else acc[r] + t
            svec = sv[pl.ds(o, L)]
            bvec = bv[pl.ds(o, L)]
            for r in range(RB):
                z = acc[r] * svec - bvec
                yr[r][pl.ds(o, L)] = 1.0 / (1.0 + jnp.exp(-z))
            return c2

        lax.fori_loop(0, G, grp_body, 0)
        ocps = [pltpu.async_copy(yr[r], out_hbm.at[row + r], sem)
                for r in range(RB)]
        for c in ocps:
            c.wait()
        return carry

    lax.fori_loop(0, NPASS, pass_body, 0)


def kernel(x, idx, w, theta, s_raw):
    idxT = jnp.asarray(idx, jnp.int32).T
    wT = w.T
    s = jax.nn.softplus(s_raw) + 1e-6
    bterm = s * theta

    mesh = plsc.VectorSubcoreMesh(core_axis_name="c", subcore_axis_name="s")
    run = functools.partial(
        pl.kernel,
        mesh=mesh,
        compiler_params=pltpu.CompilerParams(needs_layout_passes=False),
        out_type=jax.ShapeDtypeStruct((B, OUT_DIM), jnp.float32),
        scratch_types=(
            [pltpu.VMEM((IN_DIM,), jnp.float32) for _ in range(RB)]
            + [pltpu.VMEM((OUT_DIM,), jnp.float32) for _ in range(RB)]
            + [
                pltpu.VMEM((FAN_IN, OUT_DIM), jnp.int32),
                pltpu.VMEM((FAN_IN, OUT_DIM), jnp.float32),
                pltpu.VMEM((OUT_DIM,), jnp.float32),
                pltpu.VMEM((OUT_DIM,), jnp.float32),
                pltpu.SemaphoreType.DMA,
            ]
        ),
    )(_tec_body)
    return run(x, idxT, wT, s, bterm)
